# Initial kernel scaffold; baseline (speedup 1.0000x reference)
import jax
import jax.numpy as jnp
from jax.experimental import pallas as pl


def kernel(x, weights, edge_src, edge_dst, edge_rel):
    raise NotImplementedError("write your pallas kernel here")


# speedup vs baseline: 4.9996x; 4.9996x over previous
import functools

import jax
import jax.numpy as jnp
from jax import lax
from jax.experimental import pallas as pl
from jax.experimental.pallas import tpu as pltpu
from jax.experimental.pallas import tpu_sc as plsc

N = 10000
R = 8
E = 320000
EMB = 128
NC = 2
NS = 16
CHUNK = 80
CPAD = NS * 5120


def _mm_body(x_ref, w_ref, y_ref):
    y_ref[0] = lax.dot_general(
        x_ref[...], w_ref[0],
        dimension_numbers=(((1,), (1,)), ((), ())),
        preferred_element_type=jnp.float32)


_BN = 2000
_mm = pl.pallas_call(
    _mm_body,
    grid=(N // _BN, R),
    in_specs=[
        pl.BlockSpec((_BN, EMB), lambda i, r: (i, 0)),
        pl.BlockSpec((1, EMB, EMB), lambda i, r: (r, 0, 0)),
    ],
    out_specs=pl.BlockSpec((1, _BN, EMB), lambda i, r: (r, i, 0)),
    out_shape=jax.ShapeDtypeStruct((R, N, EMB), jnp.float32),
)


def _sc_body(y_hbm, src_hbm, dst_hbm, rel_hbm, out_hbm,
             src_v, dst_v, rel_v, yidx_v, cidx_v, ones_v, cval_v,
             rows_v, zero1_v, zrows_v, counts_s, acc_s, sem_y, sem_c):
    cid = lax.axis_index("c")
    sid = lax.axis_index("s")
    zero16 = jnp.zeros((16,), jnp.float32)
    ones16 = jnp.ones((16,), jnp.float32)

    def _z1(i, c):
        zero1_v[pl.ds(i * 16, 16)] = zero16
        return c
    lax.fori_loop(0, 5120 // 16, _z1, 0)

    def _z2(j, c):
        for k in range(EMB // 16):
            zrows_v[j, pl.ds(k * 16, 16)] = zero16
        return c
    lax.fori_loop(0, 125, _z2, 0)

    for j in range(CHUNK // 16):
        ones_v[pl.ds(j * 16, 16)] = ones16

    pltpu.sync_copy(zero1_v, counts_s.at[pl.ds(sid * 5120, 5120)])
    for t in range(5):
        pltpu.sync_copy(zrows_v, acc_s.at[pl.ds(sid * 625 + t * 125, 125)])
    plsc.subcore_barrier()

    eps = E // NS
    base1 = sid * eps

    def _count_step(g, c):
        b = base1 + g * CHUNK
        pltpu.sync_copy(rel_hbm.at[pl.ds(b, CHUNK)], rel_v)
        pltpu.sync_copy(dst_hbm.at[pl.ds(b, CHUNK)], dst_v)
        for j in range(CHUNK // 16):
            r16 = rel_v[pl.ds(j * 16, 16)]
            d16 = dst_v[pl.ds(j * 16, 16)]
            cidx_v[pl.ds(j * 16, 16)] = r16 * N + d16
        pltpu.sync_copy(ones_v, counts_s.at[cidx_v], add=True)
        return c
    lax.fori_loop(0, eps // CHUNK, _count_step, 0)
    plsc.subcore_barrier()

    eps2 = E // (NC * NS)
    base2 = cid * (E // NC) + sid * eps2

    def _msg_step(g, c):
        b = base2 + g * CHUNK
        pltpu.sync_copy(src_hbm.at[pl.ds(b, CHUNK)], src_v)
        pltpu.sync_copy(rel_hbm.at[pl.ds(b, CHUNK)], rel_v)
        pltpu.sync_copy(dst_hbm.at[pl.ds(b, CHUNK)], dst_v)
        for j in range(CHUNK // 16):
            r16 = rel_v[pl.ds(j * 16, 16)]
            s16 = src_v[pl.ds(j * 16, 16)]
            d16 = dst_v[pl.ds(j * 16, 16)]
            yidx_v[pl.ds(j * 16, 16)] = r16 * N + s16
            cidx_v[pl.ds(j * 16, 16)] = r16 * N + d16
        cp_y = pltpu.async_copy(y_hbm.at[yidx_v], rows_v, sem_y)
        cp_c = pltpu.async_copy(counts_s.at[cidx_v], cval_v, sem_c)
        cp_c.wait()
        for j in range(CHUNK // 16):
            c16 = cval_v[pl.ds(j * 16, 16)]
            cval_v[pl.ds(j * 16, 16)] = 1.0 / c16
        cp_y.wait()

        def _scale(j, cc):
            val16 = cval_v[pl.ds(j * 16, 16)]
            for l in range(16):
                v = val16[l]
                e = j * 16 + l
                for k in range(EMB // 16):
                    rows_v[e, pl.ds(k * 16, 16)] = (
                        rows_v[e, pl.ds(k * 16, 16)] * v)
            return cc
        lax.fori_loop(0, CHUNK // 16, _scale, 0)
        pltpu.sync_copy(rows_v, acc_s.at[dst_v], add=True)
        return c
    lax.fori_loop(0, eps2 // CHUNK, _msg_step, 0)
    plsc.subcore_barrier()

    pltpu.sync_copy(acc_s.at[pl.ds(sid * 625, 625)], out_hbm.at[cid, sid])


_sc_scatter = pl.kernel(
    _sc_body,
    out_type=jax.ShapeDtypeStruct((NC, NS, N // NS, EMB), jnp.float32),
    mesh=plsc.VectorSubcoreMesh(
        core_axis_name="c", subcore_axis_name="s",
        num_cores=NC, num_subcores=NS),
    scratch_types=[
        pltpu.VMEM((CHUNK,), jnp.int32),
        pltpu.VMEM((CHUNK,), jnp.int32),
        pltpu.VMEM((CHUNK,), jnp.int32),
        pltpu.VMEM((CHUNK,), jnp.int32),
        pltpu.VMEM((CHUNK,), jnp.int32),
        pltpu.VMEM((CHUNK,), jnp.float32),
        pltpu.VMEM((CHUNK,), jnp.float32),
        pltpu.VMEM((CHUNK, EMB), jnp.float32),
        pltpu.VMEM((5120,), jnp.float32),
        pltpu.VMEM((125, EMB), jnp.float32),
        pltpu.VMEM_SHARED((CPAD,), jnp.float32),
        pltpu.VMEM_SHARED((N, EMB), jnp.float32),
        pltpu.SemaphoreType.DMA,
        pltpu.SemaphoreType.DMA,
    ],
)


def _comb_body(p_ref, o_ref):
    o_ref[...] = jnp.maximum(p_ref[0] + p_ref[1], 0.0)


_comb = pl.pallas_call(
    _comb_body,
    grid=(N // _BN,),
    in_specs=[pl.BlockSpec((NC, _BN, EMB), lambda i: (0, i, 0))],
    out_specs=pl.BlockSpec((_BN, EMB), lambda i: (i, 0)),
    out_shape=jax.ShapeDtypeStruct((N, EMB), jnp.float32),
)


def kernel(x, weights, edge_src, edge_dst, edge_rel):
    edge_src = edge_src.astype(jnp.int32)
    edge_dst = edge_dst.astype(jnp.int32)
    edge_rel = edge_rel.astype(jnp.int32)
    y = _mm(x, weights).reshape(R * N, EMB)
    partial = _sc_scatter(y, edge_src, edge_dst, edge_rel)
    return _comb(partial.reshape(NC, N, EMB))


# speedup vs baseline: 17.4331x; 3.4869x over previous
import jax
import jax.numpy as jnp
from jax import lax
from jax.experimental import pallas as pl
from jax.experimental.pallas import tpu as pltpu
from jax.experimental.pallas import tpu_sc as plsc

N = 10000
R = 8
E = 320000
EMB = 128
NC = 2
NS = 16
SUB = 80
MEGA = 400
NSM = MEGA // SUB
EPT = E // (NC * NS)
NM2 = EPT // MEGA
EPT1 = E // NS
NM1 = EPT1 // MEGA
NSUBT = EPT // SUB
ROWS_N = N // NS
CSLICE = 5120


def _mm_body(x_ref, w_ref, y_ref):
    y_ref[0] = lax.dot_general(
        x_ref[...], w_ref[0],
        dimension_numbers=(((1,), (1,)), ((), ())),
        preferred_element_type=jnp.float32)


_BN = 2000
_mm = pl.pallas_call(
    _mm_body,
    grid=(N // _BN, R),
    in_specs=[
        pl.BlockSpec((_BN, EMB), lambda i, r: (i, 0)),
        pl.BlockSpec((1, EMB, EMB), lambda i, r: (r, 0, 0)),
    ],
    out_specs=pl.BlockSpec((1, _BN, EMB), lambda i, r: (r, i, 0)),
    out_shape=jax.ShapeDtypeStruct((R, N, EMB), jnp.float32),
)


def _sc_body(y_hbm, src_hbm, dst_hbm, rel_hbm, out_hbm,
             src_m, dst_m, rel_m, yidx2, cidx2, dst2,
             rows_v, cval2, ones_v, zero1_v, zrows_v,
             counts_s, acc_s,
             sem_e0, sem_e1,
             sem_y0, sem_y1, sem_y2,
             sem_c0, sem_c1, sem_c2,
             sem_s0, sem_s1, sem_s2,
             sem_p0, sem_p1, sem_z):
    cid = lax.axis_index("c")
    sid = lax.axis_index("s")
    zero16 = jnp.zeros((16,), jnp.float32)
    ones16 = jnp.ones((16,), jnp.float32)
    SEM_E = (sem_e0, sem_e1)
    SEM_Y = (sem_y0, sem_y1, sem_y2)
    SEM_C = (sem_c0, sem_c1, sem_c2)
    SEM_S = (sem_s0, sem_s1, sem_s2)
    SEM_P = (sem_p0, sem_p1)

    base1 = sid * EPT1
    base2 = cid * (E // NC) + sid * EPT

    def _z1(i, c):
        zero1_v[pl.ds(i * 16, 16)] = zero16
        return c
    lax.fori_loop(0, 1280 // 16, _z1, 0)

    def _z2(j, c):
        for k in range(EMB // 16):
            zrows_v[j, pl.ds(k * 16, 16)] = zero16
        return c
    lax.fori_loop(0, 25, _z2, 0)

    for j in range(SUB // 16):
        ones_v[pl.ds(j * 16, 16)] = ones16

    for t in range(CSLICE // 1280):
        pltpu.sync_copy(zero1_v,
                        counts_s.at[pl.ds(sid * CSLICE + t * 1280, 1280)])
    for t in range(ROWS_N // 25):
        pltpu.async_copy(zrows_v, acc_s.at[pl.ds(sid * ROWS_N + t * 25, 25)],
                         sem_z)
    pltpu.async_copy(rel_hbm.at[pl.ds(base1, MEGA)],
                     rel_m.at[pl.ds(0, MEGA)], sem_e0)
    pltpu.async_copy(dst_hbm.at[pl.ds(base1, MEGA)],
                     dst_m.at[pl.ds(0, MEGA)], sem_e0)
    plsc.subcore_barrier()

    def _drain_e1(bm, sem):
        pltpu.make_async_copy(rel_hbm.at[pl.ds(0, MEGA)],
                              rel_m.at[pl.ds(bm * MEGA, MEGA)], sem).wait()
        pltpu.make_async_copy(dst_hbm.at[pl.ds(0, MEGA)],
                              dst_m.at[pl.ds(bm * MEGA, MEGA)], sem).wait()

    def _cidx_compute(bm):
        def _g(j2, c):
            for k in range(SUB // 16):
                sl = pl.ds(bm * MEGA + j2 * SUB + k * 16, 16)
                r16 = rel_m[sl]
                d16 = dst_m[sl]
                cidx2[bm, j2, pl.ds(k * 16, 16)] = r16 * N + d16
            return c
        lax.fori_loop(0, NSM, _g, 0)

    def _drain_p(sem):
        def _w(i, c):
            pltpu.make_async_copy(ones_v, counts_s.at[cidx2.at[0, 0]],
                                  sem).wait()
            return c
        lax.fori_loop(0, NSM, _w, 0)

    def _phase1_mega(bm):
        _drain_e1(bm, SEM_E[bm])
        _cidx_compute(bm)

        def _fire(j2, c):
            pltpu.async_copy(ones_v, counts_s.at[cidx2.at[bm, j2]],
                             SEM_P[bm], add=True)
            return c
        lax.fori_loop(0, NSM, _fire, 0)

    def _t_body(t, c):
        m0 = 2 * t
        b1 = base1 + (m0 + 1) * MEGA
        pltpu.async_copy(rel_hbm.at[pl.ds(b1, MEGA)],
                         rel_m.at[pl.ds(MEGA, MEGA)], sem_e1)
        pltpu.async_copy(dst_hbm.at[pl.ds(b1, MEGA)],
                         dst_m.at[pl.ds(MEGA, MEGA)], sem_e1)

        @pl.when(t >= 1)
        def _():
            _drain_p(sem_p0)
        _phase1_mega(0)

        @pl.when(t <= NM1 // 2 - 2)
        def _():
            b2 = base1 + (m0 + 2) * MEGA
            pltpu.async_copy(rel_hbm.at[pl.ds(b2, MEGA)],
                             rel_m.at[pl.ds(0, MEGA)], sem_e0)
            pltpu.async_copy(dst_hbm.at[pl.ds(b2, MEGA)],
                             dst_m.at[pl.ds(0, MEGA)], sem_e0)

        @pl.when(t >= 1)
        def _():
            _drain_p(sem_p1)
        _phase1_mega(1)
        return c
    lax.fori_loop(0, NM1 // 2, _t_body, 0)
    _drain_p(sem_p0)
    _drain_p(sem_p1)

    for t in range(ROWS_N // 25):
        pltpu.make_async_copy(
            zrows_v, acc_s.at[pl.ds(sid * ROWS_N + t * 25, 25)],
            sem_z).wait()
    plsc.subcore_barrier()

    def _fire_e2(mg, bm, sem):
        b = base2 + mg * MEGA
        msl = pl.ds(bm * MEGA, MEGA)
        pltpu.async_copy(src_hbm.at[pl.ds(b, MEGA)], src_m.at[msl], sem)
        pltpu.async_copy(rel_hbm.at[pl.ds(b, MEGA)], rel_m.at[msl], sem)
        pltpu.async_copy(dst_hbm.at[pl.ds(b, MEGA)], dst_m.at[msl], sem)

    def _drain_e2(bm, sem):
        msl = pl.ds(bm * MEGA, MEGA)
        for hbm, mb in ((src_hbm, src_m), (rel_hbm, rel_m), (dst_hbm, dst_m)):
            pltpu.make_async_copy(hbm.at[pl.ds(0, MEGA)], mb.at[msl],
                                  sem).wait()

    def _idx_compute(bm):
        def _g(j2, c):
            for k in range(SUB // 16):
                sl = pl.ds(bm * MEGA + j2 * SUB + k * 16, 16)
                r16 = rel_m[sl]
                s16 = src_m[sl]
                d16 = dst_m[sl]
                rN = r16 * N
                ksl = pl.ds(k * 16, 16)
                yidx2[bm, j2, ksl] = rN + s16
                cidx2[bm, j2, ksl] = rN + d16
                dst2[bm, j2, ksl] = d16
            return c
        lax.fori_loop(0, NSM, _g, 0)

    def _fire_g(jg, p):
        bmg = (jg // NSM) % 2
        jj = jg % NSM
        pltpu.async_copy(y_hbm.at[yidx2.at[bmg, jj]], rows_v.at[p], SEM_Y[p])
        pltpu.async_copy(counts_s.at[cidx2.at[bmg, jj]], cval2.at[p],
                         SEM_C[p])

    def _wait_g(p):
        pltpu.make_async_copy(y_hbm.at[yidx2.at[0, 0]], rows_v.at[p],
                              SEM_Y[p]).wait()
        pltpu.make_async_copy(counts_s.at[cidx2.at[0, 0]], cval2.at[p],
                              SEM_C[p]).wait()

    def _wait_s(p):
        pltpu.make_async_copy(rows_v.at[p], acc_s.at[dst2.at[0, 0]],
                              SEM_S[p]).wait()

    def _process(j, p):
        _wait_g(p)

        def _sc16(g, c):
            val16 = 1.0 / cval2[p, pl.ds(g * 16, 16)]
            for l in range(16):
                v = val16[l]
                e = g * 16 + l
                for k in range(EMB // 16):
                    rows_v[p, e, pl.ds(k * 16, 16)] = (
                        rows_v[p, e, pl.ds(k * 16, 16)] * v)
            return c
        lax.fori_loop(0, SUB // 16, _sc16, 0)
        bm = (j // NSM) % 2
        jj = j % NSM
        pltpu.async_copy(rows_v.at[p], acc_s.at[dst2.at[bm, jj]],
                         SEM_S[p], add=True)

    def _boundary(j):
        m1 = (j + 1) // NSM
        bm1 = m1 % 2

        @pl.when(bm1 == 0)
        def _():
            _drain_e2(0, sem_e0)

        @pl.when(bm1 == 1)
        def _():
            _drain_e2(1, sem_e1)
        _idx_compute(bm1)

        @pl.when(m1 + 1 <= NM2 - 1)
        def _():
            @pl.when(bm1 == 1)
            def _():
                _fire_e2(m1 + 1, 0, sem_e0)

            @pl.when(bm1 == 0)
            def _():
                _fire_e2(m1 + 1, 1, sem_e1)

    _fire_e2(0, 0, sem_e0)
    _drain_e2(0, sem_e0)
    _idx_compute(0)
    _fire_e2(1, 1, sem_e1)
    _fire_g(0, 0)

    def _q_body(q, c):
        for p3 in range(3):
            j = 3 * q + p3

            @pl.when((j % NSM == NSM - 1) & (j <= NSUBT - 6))
            def _():
                _boundary(j)

            pn = (p3 + 1) % 3

            @pl.when(j >= 2)
            def _():
                _wait_s(pn)
            _fire_g(j + 1, pn)
            _process(j, p3)
        return c
    lax.fori_loop(0, (NSUBT - 2) // 3, _q_body, 0)
    _wait_s(1)
    _fire_g(NSUBT - 1, 1)
    _process(NSUBT - 2, 0)
    _process(NSUBT - 1, 1)
    for p in (2, 0, 1):
        _wait_s(p)
    plsc.subcore_barrier()

    pltpu.sync_copy(acc_s.at[pl.ds(sid * ROWS_N, ROWS_N)],
                    out_hbm.at[cid, sid])


_sc_scatter = pl.kernel(
    _sc_body,
    out_type=jax.ShapeDtypeStruct((NC, NS, ROWS_N, EMB), jnp.float32),
    mesh=plsc.VectorSubcoreMesh(
        core_axis_name="c", subcore_axis_name="s",
        num_cores=NC, num_subcores=NS),
    scratch_types=[
        pltpu.VMEM((2 * MEGA,), jnp.int32),
        pltpu.VMEM((2 * MEGA,), jnp.int32),
        pltpu.VMEM((2 * MEGA,), jnp.int32),
        pltpu.VMEM((2, NSM, SUB), jnp.int32),
        pltpu.VMEM((2, NSM, SUB), jnp.int32),
        pltpu.VMEM((2, NSM, SUB), jnp.int32),
        pltpu.VMEM((3, SUB, EMB), jnp.float32),
        pltpu.VMEM((3, SUB), jnp.float32),
        pltpu.VMEM((SUB,), jnp.float32),
        pltpu.VMEM((1280,), jnp.float32),
        pltpu.VMEM((25, EMB), jnp.float32),
        pltpu.VMEM_SHARED((NS * CSLICE,), jnp.float32),
        pltpu.VMEM_SHARED((N, EMB), jnp.float32),
    ] + [pltpu.SemaphoreType.DMA] * 14,
)


def _comb_body(p_ref, o_ref):
    o_ref[...] = jnp.maximum(p_ref[0] + p_ref[1], 0.0)


_comb = pl.pallas_call(
    _comb_body,
    grid=(N // _BN,),
    in_specs=[pl.BlockSpec((NC, _BN, EMB), lambda i: (0, i, 0))],
    out_specs=pl.BlockSpec((_BN, EMB), lambda i: (i, 0)),
    out_shape=jax.ShapeDtypeStruct((N, EMB), jnp.float32),
)


def kernel(x, weights, edge_src, edge_dst, edge_rel):
    edge_src = edge_src.astype(jnp.int32)
    edge_dst = edge_dst.astype(jnp.int32)
    edge_rel = edge_rel.astype(jnp.int32)
    y = _mm(x, weights).reshape(R * N, EMB)
    partial = _sc_scatter(y, edge_src, edge_dst, edge_rel)
    return _comb(partial.reshape(NC, N, EMB))


# speedup vs baseline: 18.2800x; 1.0486x over previous
import jax
import jax.numpy as jnp
from jax import lax
from jax.experimental import pallas as pl
from jax.experimental.pallas import tpu as pltpu
from jax.experimental.pallas import tpu_sc as plsc

N = 10000
R = 8
E = 320000
EMB = 128
NC = 2
NS = 16
SUB = 80
MEGA = 400
NSM = MEGA // SUB
EPT = E // (NC * NS)
NM = EPT // MEGA
NSUBT = EPT // SUB
ROWS_N = N // NS
CSLICE = 5120
CPAD = NS * CSLICE


def _mm_body(x_ref, w_ref, y_ref):
    y_ref[0] = lax.dot_general(
        x_ref[...], w_ref[0],
        dimension_numbers=(((1,), (1,)), ((), ())),
        preferred_element_type=jnp.float32)


_BN = 2000
_mm = pl.pallas_call(
    _mm_body,
    grid=(N // _BN, R),
    in_specs=[
        pl.BlockSpec((_BN, EMB), lambda i, r: (i, 0)),
        pl.BlockSpec((1, EMB, EMB), lambda i, r: (r, 0, 0)),
    ],
    out_specs=pl.BlockSpec((1, _BN, EMB), lambda i, r: (r, i, 0)),
    out_shape=jax.ShapeDtypeStruct((R, N, EMB), jnp.float32),
)


def _counts_body(dst_hbm, rel_hbm, cnt_hbm,
                 dst_m, rel_m, cidx2, ones_v, zero1_v, counts_s,
                 sem_e0, sem_e1, sem_p0, sem_p1):
    cid = lax.axis_index("c")
    sid = lax.axis_index("s")
    zero16 = jnp.zeros((16,), jnp.float32)
    ones16 = jnp.ones((16,), jnp.float32)
    SEM_E = (sem_e0, sem_e1)
    SEM_P = (sem_p0, sem_p1)
    base = cid * (E // NC) + sid * EPT

    def _z1(i, c):
        zero1_v[pl.ds(i * 16, 16)] = zero16
        return c
    lax.fori_loop(0, 1280 // 16, _z1, 0)
    for j in range(SUB // 16):
        ones_v[pl.ds(j * 16, 16)] = ones16
    for t in range(CSLICE // 1280):
        pltpu.sync_copy(zero1_v,
                        counts_s.at[pl.ds(sid * CSLICE + t * 1280, 1280)])
    pltpu.async_copy(rel_hbm.at[pl.ds(base, MEGA)],
                     rel_m.at[pl.ds(0, MEGA)], sem_e0)
    pltpu.async_copy(dst_hbm.at[pl.ds(base, MEGA)],
                     dst_m.at[pl.ds(0, MEGA)], sem_e0)
    plsc.subcore_barrier()

    def _drain_e(bm, sem):
        pltpu.make_async_copy(rel_hbm.at[pl.ds(0, MEGA)],
                              rel_m.at[pl.ds(bm * MEGA, MEGA)], sem).wait()
        pltpu.make_async_copy(dst_hbm.at[pl.ds(0, MEGA)],
                              dst_m.at[pl.ds(bm * MEGA, MEGA)], sem).wait()

    def _cidx_compute(bm):
        def _g(j2, c):
            for k in range(SUB // 16):
                sl = pl.ds(bm * MEGA + j2 * SUB + k * 16, 16)
                cidx2[bm, j2, pl.ds(k * 16, 16)] = rel_m[sl] * N + dst_m[sl]
            return c
        lax.fori_loop(0, NSM, _g, 0)

    def _drain_p(sem):
        def _w(i, c):
            pltpu.make_async_copy(ones_v, counts_s.at[cidx2.at[0, 0]],
                                  sem).wait()
            return c
        lax.fori_loop(0, NSM, _w, 0)

    def _mega(bm):
        _drain_e(bm, SEM_E[bm])
        _cidx_compute(bm)

        def _fire(j2, c):
            pltpu.async_copy(ones_v, counts_s.at[cidx2.at[bm, j2]],
                             SEM_P[bm], add=True)
            return c
        lax.fori_loop(0, NSM, _fire, 0)

    def _t_body(t, c):
        m0 = 2 * t
        b1 = base + (m0 + 1) * MEGA
        pltpu.async_copy(rel_hbm.at[pl.ds(b1, MEGA)],
                         rel_m.at[pl.ds(MEGA, MEGA)], sem_e1)
        pltpu.async_copy(dst_hbm.at[pl.ds(b1, MEGA)],
                         dst_m.at[pl.ds(MEGA, MEGA)], sem_e1)

        @pl.when(t >= 1)
        def _():
            _drain_p(sem_p0)
        _mega(0)

        b2 = base + (m0 + 2) * MEGA
        pltpu.async_copy(rel_hbm.at[pl.ds(b2, MEGA)],
                         rel_m.at[pl.ds(0, MEGA)], sem_e0)
        pltpu.async_copy(dst_hbm.at[pl.ds(b2, MEGA)],
                         dst_m.at[pl.ds(0, MEGA)], sem_e0)

        @pl.when(t >= 1)
        def _():
            _drain_p(sem_p1)
        _mega(1)
        return c
    lax.fori_loop(0, NM // 2, _t_body, 0)
    _drain_p(sem_p0)
    _mega(0)
    _drain_p(sem_p0)
    _drain_p(sem_p1)
    plsc.subcore_barrier()
    pltpu.sync_copy(counts_s.at[pl.ds(sid * CSLICE, CSLICE)],
                    cnt_hbm.at[pl.ds(cid * CPAD + sid * CSLICE, CSLICE)])


_sc_counts = pl.kernel(
    _counts_body,
    out_type=jax.ShapeDtypeStruct((NC * CPAD,), jnp.float32),
    mesh=plsc.VectorSubcoreMesh(
        core_axis_name="c", subcore_axis_name="s",
        num_cores=NC, num_subcores=NS),
    scratch_types=[
        pltpu.VMEM((2 * MEGA,), jnp.int32),
        pltpu.VMEM((2 * MEGA,), jnp.int32),
        pltpu.VMEM((2, NSM, SUB), jnp.int32),
        pltpu.VMEM((SUB,), jnp.float32),
        pltpu.VMEM((1280,), jnp.float32),
        pltpu.VMEM_SHARED((CPAD,), jnp.float32),
    ] + [pltpu.SemaphoreType.DMA] * 4,
)


def _sc_body(y_hbm, src_hbm, dst_hbm, rel_hbm, cnt_hbm, out_hbm,
             src_m, dst_m, rel_m, yidx2, cidx2, dst2,
             rows_v, cval2, ca_v, cb_v, zrows_v,
             counts_s, acc_s,
             sem_e0, sem_e1,
             sem_y0, sem_y1, sem_y2,
             sem_c0, sem_c1, sem_c2,
             sem_s0, sem_s1, sem_s2,
             sem_z):
    cid = lax.axis_index("c")
    sid = lax.axis_index("s")
    zero16 = jnp.zeros((16,), jnp.float32)
    SEM_E = (sem_e0, sem_e1)
    SEM_Y = (sem_y0, sem_y1, sem_y2)
    SEM_C = (sem_c0, sem_c1, sem_c2)
    SEM_S = (sem_s0, sem_s1, sem_s2)

    base2 = cid * (E // NC) + sid * EPT

    def _z2(j, c):
        for k in range(EMB // 16):
            zrows_v[j, pl.ds(k * 16, 16)] = zero16
        return c
    lax.fori_loop(0, 25, _z2, 0)
    for t in range(ROWS_N // 25):
        pltpu.async_copy(zrows_v, acc_s.at[pl.ds(sid * ROWS_N + t * 25, 25)],
                         sem_z)
    def _stage(t, c):
        off = sid * CSLICE + t * 640
        pltpu.async_copy(cnt_hbm.at[pl.ds(off, 640)], ca_v, sem_e0)
        pltpu.async_copy(cnt_hbm.at[pl.ds(CPAD + off, 640)], cb_v, sem_e1)
        pltpu.make_async_copy(cnt_hbm.at[pl.ds(0, 640)], ca_v, sem_e0).wait()
        pltpu.make_async_copy(cnt_hbm.at[pl.ds(0, 640)], cb_v, sem_e1).wait()

        def _add(i, cc):
            sl = pl.ds(i * 16, 16)
            ca_v[sl] = ca_v[sl] + cb_v[sl]
            return cc
        lax.fori_loop(0, 640 // 16, _add, 0)
        pltpu.sync_copy(ca_v, counts_s.at[pl.ds(off, 640)])
        return c
    lax.fori_loop(0, CSLICE // 640, _stage, 0)
    for t in range(ROWS_N // 25):
        pltpu.make_async_copy(
            zrows_v, acc_s.at[pl.ds(sid * ROWS_N + t * 25, 25)],
            sem_z).wait()
    plsc.subcore_barrier()

    def _fire_e2(mg, bm, sem):
        b = base2 + mg * MEGA
        msl = pl.ds(bm * MEGA, MEGA)
        pltpu.async_copy(src_hbm.at[pl.ds(b, MEGA)], src_m.at[msl], sem)
        pltpu.async_copy(rel_hbm.at[pl.ds(b, MEGA)], rel_m.at[msl], sem)
        pltpu.async_copy(dst_hbm.at[pl.ds(b, MEGA)], dst_m.at[msl], sem)

    def _drain_e2(bm, sem):
        msl = pl.ds(bm * MEGA, MEGA)
        for hbm, mb in ((src_hbm, src_m), (rel_hbm, rel_m), (dst_hbm, dst_m)):
            pltpu.make_async_copy(hbm.at[pl.ds(0, MEGA)], mb.at[msl],
                                  sem).wait()

    def _idx_compute(bm):
        def _g(j2, c):
            for k in range(SUB // 16):
                sl = pl.ds(bm * MEGA + j2 * SUB + k * 16, 16)
                r16 = rel_m[sl]
                s16 = src_m[sl]
                d16 = dst_m[sl]
                rN = r16 * N
                ksl = pl.ds(k * 16, 16)
                yidx2[bm, j2, ksl] = rN + s16
                cidx2[bm, j2, ksl] = rN + d16
                dst2[bm, j2, ksl] = d16
            return c
        lax.fori_loop(0, NSM, _g, 0)

    def _fire_g(jg, p):
        bmg = (jg // NSM) % 2
        jj = jg % NSM
        pltpu.async_copy(y_hbm.at[yidx2.at[bmg, jj]], rows_v.at[p], SEM_Y[p])
        pltpu.async_copy(counts_s.at[cidx2.at[bmg, jj]], cval2.at[p],
                         SEM_C[p])

    def _wait_g(p):
        pltpu.make_async_copy(y_hbm.at[yidx2.at[0, 0]], rows_v.at[p],
                              SEM_Y[p]).wait()
        pltpu.make_async_copy(counts_s.at[cidx2.at[0, 0]], cval2.at[p],
                              SEM_C[p]).wait()

    def _wait_s(p):
        pltpu.make_async_copy(rows_v.at[p], acc_s.at[dst2.at[0, 0]],
                              SEM_S[p]).wait()

    def _process(j, p):
        _wait_g(p)

        def _sc16(g, c):
            val16 = 1.0 / cval2[p, pl.ds(g * 16, 16)]
            for l in range(16):
                v = val16[l]
                e = g * 16 + l
                for k in range(EMB // 16):
                    rows_v[p, e, pl.ds(k * 16, 16)] = (
                        rows_v[p, e, pl.ds(k * 16, 16)] * v)
            return c
        lax.fori_loop(0, SUB // 16, _sc16, 0)
        bm = (j // NSM) % 2
        jj = j % NSM
        pltpu.async_copy(rows_v.at[p], acc_s.at[dst2.at[bm, jj]],
                         SEM_S[p], add=True)

    def _boundary(j):
        m1 = (j + 1) // NSM
        bm1 = m1 % 2

        @pl.when(bm1 == 0)
        def _():
            _drain_e2(0, sem_e0)

        @pl.when(bm1 == 1)
        def _():
            _drain_e2(1, sem_e1)
        _idx_compute(bm1)

        @pl.when(m1 + 1 <= NM - 1)
        def _():
            @pl.when(bm1 == 1)
            def _():
                _fire_e2(m1 + 1, 0, sem_e0)

            @pl.when(bm1 == 0)
            def _():
                _fire_e2(m1 + 1, 1, sem_e1)

    _fire_e2(0, 0, sem_e0)
    _drain_e2(0, sem_e0)
    _idx_compute(0)
    _fire_e2(1, 1, sem_e1)
    _fire_g(0, 0)

    def _q_body(q, c):
        for p3 in range(3):
            j = 3 * q + p3

            @pl.when((j % NSM == NSM - 1) & (j <= NSUBT - 6))
            def _():
                _boundary(j)

            pn = (p3 + 1) % 3

            @pl.when(j >= 2)
            def _():
                _wait_s(pn)
            _fire_g(j + 1, pn)
            _process(j, p3)
        return c
    lax.fori_loop(0, (NSUBT - 2) // 3, _q_body, 0)
    _wait_s(1)
    _fire_g(NSUBT - 1, 1)
    _process(NSUBT - 2, 0)
    _process(NSUBT - 1, 1)
    for p in (2, 0, 1):
        _wait_s(p)
    plsc.subcore_barrier()

    pltpu.sync_copy(acc_s.at[pl.ds(sid * ROWS_N, ROWS_N)],
                    out_hbm.at[cid, sid])


_sc_scatter = pl.kernel(
    _sc_body,
    out_type=jax.ShapeDtypeStruct((NC, NS, ROWS_N, EMB), jnp.float32),
    mesh=plsc.VectorSubcoreMesh(
        core_axis_name="c", subcore_axis_name="s",
        num_cores=NC, num_subcores=NS),
    scratch_types=[
        pltpu.VMEM((2 * MEGA,), jnp.int32),
        pltpu.VMEM((2 * MEGA,), jnp.int32),
        pltpu.VMEM((2 * MEGA,), jnp.int32),
        pltpu.VMEM((2, NSM, SUB), jnp.int32),
        pltpu.VMEM((2, NSM, SUB), jnp.int32),
        pltpu.VMEM((2, NSM, SUB), jnp.int32),
        pltpu.VMEM((3, SUB, EMB), jnp.float32),
        pltpu.VMEM((3, SUB), jnp.float32),
        pltpu.VMEM((640,), jnp.float32),
        pltpu.VMEM((640,), jnp.float32),
        pltpu.VMEM((25, EMB), jnp.float32),
        pltpu.VMEM_SHARED((CPAD,), jnp.float32),
        pltpu.VMEM_SHARED((N, EMB), jnp.float32),
    ] + [pltpu.SemaphoreType.DMA] * 12,
)


def _comb_body(p_ref, o_ref):
    o_ref[...] = jnp.maximum(p_ref[0] + p_ref[1], 0.0)


_comb = pl.pallas_call(
    _comb_body,
    grid=(N // _BN,),
    in_specs=[pl.BlockSpec((NC, _BN, EMB), lambda i: (0, i, 0))],
    out_specs=pl.BlockSpec((_BN, EMB), lambda i: (i, 0)),
    out_shape=jax.ShapeDtypeStruct((N, EMB), jnp.float32),
)


def kernel(x, weights, edge_src, edge_dst, edge_rel):
    edge_src = edge_src.astype(jnp.int32)
    edge_dst = edge_dst.astype(jnp.int32)
    edge_rel = edge_rel.astype(jnp.int32)
    counts = _sc_counts(edge_dst, edge_rel)
    y = _mm(x, weights).reshape(R * N, EMB)
    partial = _sc_scatter(y, edge_src, edge_dst, edge_rel, counts)
    return _comb(partial.reshape(NC, N, EMB))


# speedup vs baseline: 19.0800x; 1.0438x over previous
import jax
import jax.numpy as jnp
from jax import lax
from jax.experimental import pallas as pl
from jax.experimental.pallas import tpu as pltpu
from jax.experimental.pallas import tpu_sc as plsc

N = 10000
R = 8
E = 320000
EMB = 128
NC = 2
NS = 16
SUB = 80
MEGA = 400
NSM = MEGA // SUB
EPT = E // (NC * NS)
NM = EPT // MEGA
NSUBT = EPT // SUB
ROWS_N = N // NS
CSLICE = 5120
CPAD = NS * CSLICE


def _mm_body(x_ref, w_ref, y_ref):
    y_ref[0] = lax.dot_general(
        x_ref[...], w_ref[0],
        dimension_numbers=(((1,), (1,)), ((), ())),
        preferred_element_type=jnp.float32)


_BN = 2000
_mm = pl.pallas_call(
    _mm_body,
    grid=(N // _BN, R),
    in_specs=[
        pl.BlockSpec((_BN, EMB), lambda i, r: (i, 0)),
        pl.BlockSpec((1, EMB, EMB), lambda i, r: (r, 0, 0)),
    ],
    out_specs=pl.BlockSpec((1, _BN, EMB), lambda i, r: (r, i, 0)),
    out_shape=jax.ShapeDtypeStruct((R, N, EMB), jnp.float32),
)


def _counts_body(dst_hbm, rel_hbm, cnt_hbm,
                 dst_m, rel_m, cidx2, ones_v, zero1_v, counts_s,
                 sem_e0, sem_e1, sem_p0, sem_p1):
    cid = lax.axis_index("c")
    sid = lax.axis_index("s")
    zero16 = jnp.zeros((16,), jnp.float32)
    ones16 = jnp.ones((16,), jnp.float32)
    SEM_E = (sem_e0, sem_e1)
    SEM_P = (sem_p0, sem_p1)
    base = cid * (E // NC) + sid * EPT

    def _z1(i, c):
        zero1_v[pl.ds(i * 16, 16)] = zero16
        return c
    lax.fori_loop(0, 1280 // 16, _z1, 0)
    for j in range(SUB // 16):
        ones_v[pl.ds(j * 16, 16)] = ones16
    for t in range(CSLICE // 1280):
        pltpu.sync_copy(zero1_v,
                        counts_s.at[pl.ds(sid * CSLICE + t * 1280, 1280)])
    pltpu.async_copy(rel_hbm.at[pl.ds(base, MEGA)],
                     rel_m.at[pl.ds(0, MEGA)], sem_e0)
    pltpu.async_copy(dst_hbm.at[pl.ds(base, MEGA)],
                     dst_m.at[pl.ds(0, MEGA)], sem_e0)
    plsc.subcore_barrier()

    def _drain_e(bm, sem):
        pltpu.make_async_copy(rel_hbm.at[pl.ds(0, MEGA)],
                              rel_m.at[pl.ds(bm * MEGA, MEGA)], sem).wait()
        pltpu.make_async_copy(dst_hbm.at[pl.ds(0, MEGA)],
                              dst_m.at[pl.ds(bm * MEGA, MEGA)], sem).wait()

    def _cidx_compute(bm):
        def _g(j2, c):
            for k in range(SUB // 16):
                sl = pl.ds(bm * MEGA + j2 * SUB + k * 16, 16)
                cidx2[bm, j2, pl.ds(k * 16, 16)] = rel_m[sl] * N + dst_m[sl]
            return c
        lax.fori_loop(0, NSM, _g, 0)

    def _drain_p(sem):
        def _w(i, c):
            pltpu.make_async_copy(ones_v, counts_s.at[cidx2.at[0, 0]],
                                  sem).wait()
            return c
        lax.fori_loop(0, NSM, _w, 0)

    def _mega(bm):
        _drain_e(bm, SEM_E[bm])
        _cidx_compute(bm)

        def _fire(j2, c):
            pltpu.async_copy(ones_v, counts_s.at[cidx2.at[bm, j2]],
                             SEM_P[bm], add=True)
            return c
        lax.fori_loop(0, NSM, _fire, 0)

    def _t_body(t, c):
        m0 = 2 * t
        b1 = base + (m0 + 1) * MEGA
        pltpu.async_copy(rel_hbm.at[pl.ds(b1, MEGA)],
                         rel_m.at[pl.ds(MEGA, MEGA)], sem_e1)
        pltpu.async_copy(dst_hbm.at[pl.ds(b1, MEGA)],
                         dst_m.at[pl.ds(MEGA, MEGA)], sem_e1)

        @pl.when(t >= 1)
        def _():
            _drain_p(sem_p0)
        _mega(0)

        b2 = base + (m0 + 2) * MEGA
        pltpu.async_copy(rel_hbm.at[pl.ds(b2, MEGA)],
                         rel_m.at[pl.ds(0, MEGA)], sem_e0)
        pltpu.async_copy(dst_hbm.at[pl.ds(b2, MEGA)],
                         dst_m.at[pl.ds(0, MEGA)], sem_e0)

        @pl.when(t >= 1)
        def _():
            _drain_p(sem_p1)
        _mega(1)
        return c
    lax.fori_loop(0, NM // 2, _t_body, 0)
    _drain_p(sem_p0)
    _mega(0)
    _drain_p(sem_p0)
    _drain_p(sem_p1)
    plsc.subcore_barrier()
    pltpu.sync_copy(counts_s.at[pl.ds(sid * CSLICE, CSLICE)],
                    cnt_hbm.at[pl.ds(cid * CPAD + sid * CSLICE, CSLICE)])


_sc_counts = pl.kernel(
    _counts_body,
    out_type=jax.ShapeDtypeStruct((NC * CPAD,), jnp.float32),
    mesh=plsc.VectorSubcoreMesh(
        core_axis_name="c", subcore_axis_name="s",
        num_cores=NC, num_subcores=NS),
    scratch_types=[
        pltpu.VMEM((2 * MEGA,), jnp.int32),
        pltpu.VMEM((2 * MEGA,), jnp.int32),
        pltpu.VMEM((2, NSM, SUB), jnp.int32),
        pltpu.VMEM((SUB,), jnp.float32),
        pltpu.VMEM((1280,), jnp.float32),
        pltpu.VMEM_SHARED((CPAD,), jnp.float32),
    ] + [pltpu.SemaphoreType.DMA] * 4,
)


def _sc_body(y_hbm, src_hbm, dst_hbm, rel_hbm, cnt_hbm, out_hbm,
             src_m, dst_m, rel_m, yidx2, cidx2, dst2,
             rows_v, cval2, ca_v, cb_v, zrows_v,
             counts_s, acc_s,
             sem_e0, sem_e1,
             sem_y0, sem_y1, sem_y2,
             sem_c0, sem_c1, sem_c2,
             sem_s0, sem_s1, sem_s2,
             sem_z):
    cid = lax.axis_index("c")
    sid = lax.axis_index("s")
    zero16 = jnp.zeros((16,), jnp.float32)
    SEM_E = (sem_e0, sem_e1)
    SEM_Y = (sem_y0, sem_y1, sem_y2)
    SEM_C = (sem_c0, sem_c1, sem_c2)
    SEM_S = (sem_s0, sem_s1, sem_s2)

    base2 = cid * (E // NC) + sid * EPT

    def _z2(j, c):
        for k in range(EMB // 16):
            zrows_v[j, pl.ds(k * 16, 16)] = zero16
        return c
    lax.fori_loop(0, 25, _z2, 0)
    for t in range(ROWS_N // 25):
        pltpu.async_copy(zrows_v, acc_s.at[pl.ds(sid * ROWS_N + t * 25, 25)],
                         sem_z)
    def _stage(t, c):
        off = sid * CSLICE + t * 640
        pltpu.async_copy(cnt_hbm.at[pl.ds(off, 640)], ca_v, sem_e0)
        pltpu.async_copy(cnt_hbm.at[pl.ds(CPAD + off, 640)], cb_v, sem_e1)
        pltpu.make_async_copy(cnt_hbm.at[pl.ds(0, 640)], ca_v, sem_e0).wait()
        pltpu.make_async_copy(cnt_hbm.at[pl.ds(0, 640)], cb_v, sem_e1).wait()

        def _add(i, cc):
            sl = pl.ds(i * 16, 16)
            ca_v[sl] = ca_v[sl] + cb_v[sl]
            return cc
        lax.fori_loop(0, 640 // 16, _add, 0)
        pltpu.sync_copy(ca_v, counts_s.at[pl.ds(off, 640)])
        return c
    lax.fori_loop(0, CSLICE // 640, _stage, 0)
    for t in range(ROWS_N // 25):
        pltpu.make_async_copy(
            zrows_v, acc_s.at[pl.ds(sid * ROWS_N + t * 25, 25)],
            sem_z).wait()
    plsc.subcore_barrier()

    def _fire_e2(mg, bm, sem):
        b = base2 + mg * MEGA
        msl = pl.ds(bm * MEGA, MEGA)
        pltpu.async_copy(src_hbm.at[pl.ds(b, MEGA)], src_m.at[msl], sem)
        pltpu.async_copy(rel_hbm.at[pl.ds(b, MEGA)], rel_m.at[msl], sem)
        pltpu.async_copy(dst_hbm.at[pl.ds(b, MEGA)], dst_m.at[msl], sem)

    def _drain_e2(bm, sem):
        msl = pl.ds(bm * MEGA, MEGA)
        for hbm, mb in ((src_hbm, src_m), (rel_hbm, rel_m), (dst_hbm, dst_m)):
            pltpu.make_async_copy(hbm.at[pl.ds(0, MEGA)], mb.at[msl],
                                  sem).wait()

    def _idx_compute(bm):
        def _g(j2, c):
            for k in range(SUB // 16):
                sl = pl.ds(bm * MEGA + j2 * SUB + k * 16, 16)
                r16 = rel_m[sl]
                s16 = src_m[sl]
                d16 = dst_m[sl]
                rN = r16 * N
                ksl = pl.ds(k * 16, 16)
                yidx2[bm, j2, ksl] = rN + s16
                cidx2[bm, j2, ksl] = rN + d16
                dst2[bm, j2, ksl] = d16
            return c
        lax.fori_loop(0, NSM, _g, 0)

    def _fire_g(jg, p):
        bmg = (jg // NSM) % 2
        jj = jg % NSM
        pltpu.async_copy(y_hbm.at[yidx2.at[bmg, jj]], rows_v.at[p], SEM_Y[p])
        pltpu.async_copy(counts_s.at[cidx2.at[bmg, jj]], cval2.at[p],
                         SEM_C[p])

    def _wait_g(p):
        pltpu.make_async_copy(y_hbm.at[yidx2.at[0, 0]], rows_v.at[p],
                              SEM_Y[p]).wait()
        pltpu.make_async_copy(counts_s.at[cidx2.at[0, 0]], cval2.at[p],
                              SEM_C[p]).wait()

    def _wait_s(p):
        pltpu.make_async_copy(rows_v.at[p], acc_s.at[dst2.at[0, 0]],
                              SEM_S[p]).wait()

    def _process(j, p):
        _wait_g(p)

        def _sc16(g, c):
            val16 = 1.0 / cval2[p, pl.ds(g * 16, 16)]
            for l in range(16):
                v = val16[l]
                e = g * 16 + l
                for k in range(EMB // 16):
                    rows_v[p, e, pl.ds(k * 16, 16)] = (
                        rows_v[p, e, pl.ds(k * 16, 16)] * v)
            return c
        lax.fori_loop(0, SUB // 16, _sc16, 0)
        bm = (j // NSM) % 2
        jj = j % NSM
        pltpu.async_copy(rows_v.at[p], acc_s.at[dst2.at[bm, jj]],
                         SEM_S[p], add=True)

    def _boundary(j):
        m1 = (j + 1) // NSM
        bm1 = m1 % 2

        @pl.when(bm1 == 0)
        def _():
            _drain_e2(0, sem_e0)

        @pl.when(bm1 == 1)
        def _():
            _drain_e2(1, sem_e1)
        _idx_compute(bm1)

        @pl.when(m1 + 1 <= NM - 1)
        def _():
            @pl.when(bm1 == 1)
            def _():
                _fire_e2(m1 + 1, 0, sem_e0)

            @pl.when(bm1 == 0)
            def _():
                _fire_e2(m1 + 1, 1, sem_e1)

    _fire_e2(0, 0, sem_e0)
    _drain_e2(0, sem_e0)
    _idx_compute(0)
    _fire_e2(1, 1, sem_e1)
    _fire_g(0, 0)

    def _q_body(q, c):
        for p3 in range(3):
            j = 3 * q + p3

            @pl.when((j % NSM == NSM - 1) & (j <= NSUBT - 6))
            def _():
                _boundary(j)

            pn = (p3 + 1) % 3

            @pl.when(j >= 2)
            def _():
                _wait_s(pn)
            _fire_g(j + 1, pn)
            _process(j, p3)
        return c
    lax.fori_loop(0, (NSUBT - 2) // 3, _q_body, 0)
    _wait_s(1)
    _fire_g(NSUBT - 1, 1)
    _process(NSUBT - 2, 0)
    _process(NSUBT - 1, 1)
    for p in (2, 0, 1):
        _wait_s(p)
    plsc.subcore_barrier()

    pltpu.sync_copy(acc_s.at[pl.ds(sid * 624, 624)],
                    out_hbm.at[cid, pl.ds(sid * 624, 624)])

    @pl.when(sid == NS - 1)
    def _():
        pltpu.sync_copy(acc_s.at[pl.ds(NS * 624, N - NS * 624)],
                        out_hbm.at[cid, pl.ds(NS * 624, N - NS * 624)])


_sc_scatter = pl.kernel(
    _sc_body,
    out_type=jax.ShapeDtypeStruct((NC, N, EMB), jnp.float32),
    mesh=plsc.VectorSubcoreMesh(
        core_axis_name="c", subcore_axis_name="s",
        num_cores=NC, num_subcores=NS),
    scratch_types=[
        pltpu.VMEM((2 * MEGA,), jnp.int32),
        pltpu.VMEM((2 * MEGA,), jnp.int32),
        pltpu.VMEM((2 * MEGA,), jnp.int32),
        pltpu.VMEM((2, NSM, SUB), jnp.int32),
        pltpu.VMEM((2, NSM, SUB), jnp.int32),
        pltpu.VMEM((2, NSM, SUB), jnp.int32),
        pltpu.VMEM((3, SUB, EMB), jnp.float32),
        pltpu.VMEM((3, SUB), jnp.float32),
        pltpu.VMEM((640,), jnp.float32),
        pltpu.VMEM((640,), jnp.float32),
        pltpu.VMEM((25, EMB), jnp.float32),
        pltpu.VMEM_SHARED((CPAD,), jnp.float32),
        pltpu.VMEM_SHARED((N, EMB), jnp.float32),
    ] + [pltpu.SemaphoreType.DMA] * 12,
)


def _comb_body(p_ref, o_ref):
    o_ref[...] = jnp.maximum(p_ref[0] + p_ref[1], 0.0)


_comb = pl.pallas_call(
    _comb_body,
    grid=(N // _BN,),
    in_specs=[pl.BlockSpec((NC, _BN, EMB), lambda i: (0, i, 0))],
    out_specs=pl.BlockSpec((_BN, EMB), lambda i: (i, 0)),
    out_shape=jax.ShapeDtypeStruct((N, EMB), jnp.float32),
)


def kernel(x, weights, edge_src, edge_dst, edge_rel):
    edge_src = edge_src.astype(jnp.int32)
    edge_dst = edge_dst.astype(jnp.int32)
    edge_rel = edge_rel.astype(jnp.int32)
    counts = _sc_counts(edge_dst, edge_rel)
    y = _mm(x, weights).reshape(R * N, EMB)
    partial = _sc_scatter(y, edge_src, edge_dst, edge_rel, counts)
    return _comb(partial)


# speedup vs baseline: 19.5045x; 1.0222x over previous
import jax
import jax.numpy as jnp
from jax import lax
from jax.experimental import pallas as pl
from jax.experimental.pallas import tpu as pltpu
from jax.experimental.pallas import tpu_sc as plsc

N = 10000
R = 8
E = 320000
EMB = 128
NC = 2
NS = 16
SUB = 80
MEGA = 400
NSM = MEGA // SUB
EPT = E // (NC * NS)
NM = EPT // MEGA
NSUBT = EPT // SUB
ROWS_N = N // NS
CSLICE = 5120
CPAD = NS * CSLICE


def _mm_body(x_ref, w_ref, y_ref):
    y_ref[0] = lax.dot_general(
        x_ref[...], w_ref[0],
        dimension_numbers=(((1,), (1,)), ((), ())),
        preferred_element_type=jnp.float32)


_BN = 2000
_mm = pl.pallas_call(
    _mm_body,
    grid=(N // _BN, R),
    in_specs=[
        pl.BlockSpec((_BN, EMB), lambda i, r: (i, 0)),
        pl.BlockSpec((1, EMB, EMB), lambda i, r: (r, 0, 0)),
    ],
    out_specs=pl.BlockSpec((1, _BN, EMB), lambda i, r: (r, i, 0)),
    out_shape=jax.ShapeDtypeStruct((R, N, EMB), jnp.float32),
)


def _counts_body(dst_hbm, rel_hbm, cnt_hbm,
                 dst_m, rel_m, cidx2, ones_v, zero1_v, counts_s,
                 sem_e0, sem_e1, sem_p0, sem_p1):
    cid = lax.axis_index("c")
    sid = lax.axis_index("s")
    zero16 = jnp.zeros((16,), jnp.float32)
    ones16 = jnp.ones((16,), jnp.float32)
    SEM_E = (sem_e0, sem_e1)
    SEM_P = (sem_p0, sem_p1)
    base = sid * (E // NS)

    def _z1(i, c):
        zero1_v[pl.ds(i * 16, 16)] = zero16
        return c
    lax.fori_loop(0, 1280 // 16, _z1, 0)
    for j in range(SUB // 16):
        ones_v[pl.ds(j * 16, 16)] = ones16
    for t in range(CSLICE // 1280):
        pltpu.sync_copy(zero1_v,
                        counts_s.at[pl.ds(sid * CSLICE + t * 1280, 1280)])
    pltpu.async_copy(rel_hbm.at[pl.ds(base, MEGA)],
                     rel_m.at[pl.ds(0, MEGA)], sem_e0)
    pltpu.async_copy(dst_hbm.at[pl.ds(base, MEGA)],
                     dst_m.at[pl.ds(0, MEGA)], sem_e0)
    plsc.subcore_barrier()

    def _drain_e(bm, sem):
        pltpu.make_async_copy(rel_hbm.at[pl.ds(0, MEGA)],
                              rel_m.at[pl.ds(bm * MEGA, MEGA)], sem).wait()
        pltpu.make_async_copy(dst_hbm.at[pl.ds(0, MEGA)],
                              dst_m.at[pl.ds(bm * MEGA, MEGA)], sem).wait()

    def _cidx_compute(bm):
        def _g(j2, c):
            for k in range(SUB // 16):
                sl = pl.ds(bm * MEGA + j2 * SUB + k * 16, 16)
                cidx2[bm, j2, pl.ds(k * 16, 16)] = rel_m[sl] * N + dst_m[sl]
            return c
        lax.fori_loop(0, NSM, _g, 0)

    def _drain_p(sem):
        def _w(i, c):
            pltpu.make_async_copy(ones_v, counts_s.at[cidx2.at[0, 0]],
                                  sem).wait()
            return c
        lax.fori_loop(0, NSM, _w, 0)

    def _mega(bm):
        _drain_e(bm, SEM_E[bm])
        _cidx_compute(bm)

        def _fire(j2, c):
            pltpu.async_copy(ones_v, counts_s.at[cidx2.at[bm, j2]],
                             SEM_P[bm], add=True)
            return c
        lax.fori_loop(0, NSM, _fire, 0)

    NM1 = E // NS // MEGA

    def _t_body(t, c):
        m0 = 2 * t
        b1 = base + (m0 + 1) * MEGA
        pltpu.async_copy(rel_hbm.at[pl.ds(b1, MEGA)],
                         rel_m.at[pl.ds(MEGA, MEGA)], sem_e1)
        pltpu.async_copy(dst_hbm.at[pl.ds(b1, MEGA)],
                         dst_m.at[pl.ds(MEGA, MEGA)], sem_e1)

        @pl.when(t >= 1)
        def _():
            _drain_p(sem_p0)
        _mega(0)

        @pl.when(t <= NM1 // 2 - 2)
        def _():
            b2 = base + (m0 + 2) * MEGA
            pltpu.async_copy(rel_hbm.at[pl.ds(b2, MEGA)],
                             rel_m.at[pl.ds(0, MEGA)], sem_e0)
            pltpu.async_copy(dst_hbm.at[pl.ds(b2, MEGA)],
                             dst_m.at[pl.ds(0, MEGA)], sem_e0)

        @pl.when(t >= 1)
        def _():
            _drain_p(sem_p1)
        _mega(1)
        return c
    lax.fori_loop(0, NM1 // 2, _t_body, 0)
    _drain_p(sem_p0)
    _drain_p(sem_p1)
    plsc.subcore_barrier()
    pltpu.sync_copy(counts_s.at[pl.ds(sid * CSLICE, CSLICE)],
                    cnt_hbm.at[pl.ds(cid * CPAD + sid * CSLICE, CSLICE)])


_sc_counts = pl.kernel(
    _counts_body,
    out_type=jax.ShapeDtypeStruct((NC * CPAD,), jnp.float32),
    mesh=plsc.VectorSubcoreMesh(
        core_axis_name="c", subcore_axis_name="s",
        num_cores=NC, num_subcores=NS),
    scratch_types=[
        pltpu.VMEM((2 * MEGA,), jnp.int32),
        pltpu.VMEM((2 * MEGA,), jnp.int32),
        pltpu.VMEM((2, NSM, SUB), jnp.int32),
        pltpu.VMEM((SUB,), jnp.float32),
        pltpu.VMEM((1280,), jnp.float32),
        pltpu.VMEM_SHARED((CPAD,), jnp.float32),
    ] + [pltpu.SemaphoreType.DMA] * 4,
)


def _sc_body(y_hbm, src_hbm, dst_hbm, rel_hbm, cnt_hbm, out_hbm,
             src_m, dst_m, rel_m, yidx2, cidx2, dst2,
             rows_v, cval2, zrows_v,
             counts_s, acc_s,
             sem_e0, sem_e1,
             sem_y0, sem_y1, sem_y2,
             sem_c0, sem_c1, sem_c2,
             sem_s0, sem_s1, sem_s2,
             sem_z):
    cid = lax.axis_index("c")
    sid = lax.axis_index("s")
    zero16 = jnp.zeros((16,), jnp.float32)
    SEM_E = (sem_e0, sem_e1)
    SEM_Y = (sem_y0, sem_y1, sem_y2)
    SEM_C = (sem_c0, sem_c1, sem_c2)
    SEM_S = (sem_s0, sem_s1, sem_s2)

    base2 = cid * (E // NC) + sid * EPT

    def _z2(j, c):
        for k in range(EMB // 16):
            zrows_v[j, pl.ds(k * 16, 16)] = zero16
        return c
    lax.fori_loop(0, 25, _z2, 0)
    for t in range(ROWS_N // 25):
        pltpu.async_copy(zrows_v, acc_s.at[pl.ds(sid * ROWS_N + t * 25, 25)],
                         sem_z)
    off = sid * CSLICE
    pltpu.sync_copy(cnt_hbm.at[pl.ds(cid * CPAD + off, CSLICE)],
                    counts_s.at[pl.ds(off, CSLICE)])
    for t in range(ROWS_N // 25):
        pltpu.make_async_copy(
            zrows_v, acc_s.at[pl.ds(sid * ROWS_N + t * 25, 25)],
            sem_z).wait()
    plsc.subcore_barrier()

    def _fire_e2(mg, bm, sem):
        b = base2 + mg * MEGA
        msl = pl.ds(bm * MEGA, MEGA)
        pltpu.async_copy(src_hbm.at[pl.ds(b, MEGA)], src_m.at[msl], sem)
        pltpu.async_copy(rel_hbm.at[pl.ds(b, MEGA)], rel_m.at[msl], sem)
        pltpu.async_copy(dst_hbm.at[pl.ds(b, MEGA)], dst_m.at[msl], sem)

    def _drain_e2(bm, sem):
        msl = pl.ds(bm * MEGA, MEGA)
        for hbm, mb in ((src_hbm, src_m), (rel_hbm, rel_m), (dst_hbm, dst_m)):
            pltpu.make_async_copy(hbm.at[pl.ds(0, MEGA)], mb.at[msl],
                                  sem).wait()

    def _idx_compute(bm):
        def _g(j2, c):
            for k in range(SUB // 16):
                sl = pl.ds(bm * MEGA + j2 * SUB + k * 16, 16)
                r16 = rel_m[sl]
                s16 = src_m[sl]
                d16 = dst_m[sl]
                rN = r16 * N
                ksl = pl.ds(k * 16, 16)
                yidx2[bm, j2, ksl] = rN + s16
                cidx2[bm, j2, ksl] = rN + d16
                dst2[bm, j2, ksl] = d16
            return c
        lax.fori_loop(0, NSM, _g, 0)

    def _fire_g(jg, p):
        bmg = (jg // NSM) % 2
        jj = jg % NSM
        pltpu.async_copy(y_hbm.at[yidx2.at[bmg, jj]], rows_v.at[p], SEM_Y[p])
        pltpu.async_copy(counts_s.at[cidx2.at[bmg, jj]], cval2.at[p],
                         SEM_C[p])

    def _wait_g(p):
        pltpu.make_async_copy(y_hbm.at[yidx2.at[0, 0]], rows_v.at[p],
                              SEM_Y[p]).wait()
        pltpu.make_async_copy(counts_s.at[cidx2.at[0, 0]], cval2.at[p],
                              SEM_C[p]).wait()

    def _wait_s(p):
        pltpu.make_async_copy(rows_v.at[p], acc_s.at[dst2.at[0, 0]],
                              SEM_S[p]).wait()

    def _process(j, p):
        _wait_g(p)

        def _sc16(g, c):
            val16 = 1.0 / cval2[p, pl.ds(g * 16, 16)]
            for l in range(16):
                v = val16[l]
                e = g * 16 + l
                for k in range(EMB // 16):
                    rows_v[p, e, pl.ds(k * 16, 16)] = (
                        rows_v[p, e, pl.ds(k * 16, 16)] * v)
            return c
        lax.fori_loop(0, SUB // 16, _sc16, 0)
        bm = (j // NSM) % 2
        jj = j % NSM
        pltpu.async_copy(rows_v.at[p], acc_s.at[dst2.at[bm, jj]],
                         SEM_S[p], add=True)

    def _boundary(j):
        m1 = (j + 1) // NSM
        bm1 = m1 % 2

        @pl.when(bm1 == 0)
        def _():
            _drain_e2(0, sem_e0)

        @pl.when(bm1 == 1)
        def _():
            _drain_e2(1, sem_e1)
        _idx_compute(bm1)

        @pl.when(m1 + 1 <= NM - 1)
        def _():
            @pl.when(bm1 == 1)
            def _():
                _fire_e2(m1 + 1, 0, sem_e0)

            @pl.when(bm1 == 0)
            def _():
                _fire_e2(m1 + 1, 1, sem_e1)

    _fire_e2(0, 0, sem_e0)
    _drain_e2(0, sem_e0)
    _idx_compute(0)
    _fire_e2(1, 1, sem_e1)
    _fire_g(0, 0)

    def _q_body(q, c):
        for p3 in range(3):
            j = 3 * q + p3

            @pl.when((j % NSM == NSM - 1) & (j <= NSUBT - 6))
            def _():
                _boundary(j)

            pn = (p3 + 1) % 3

            @pl.when(j >= 2)
            def _():
                _wait_s(pn)
            _fire_g(j + 1, pn)
            _process(j, p3)
        return c
    lax.fori_loop(0, (NSUBT - 2) // 3, _q_body, 0)
    _wait_s(1)
    _fire_g(NSUBT - 1, 1)
    _process(NSUBT - 2, 0)
    _process(NSUBT - 1, 1)
    for p in (2, 0, 1):
        _wait_s(p)
    plsc.subcore_barrier()

    pltpu.sync_copy(acc_s.at[pl.ds(sid * 624, 624)],
                    out_hbm.at[cid, pl.ds(sid * 624, 624)])

    @pl.when(sid == NS - 1)
    def _():
        pltpu.sync_copy(acc_s.at[pl.ds(NS * 624, N - NS * 624)],
                        out_hbm.at[cid, pl.ds(NS * 624, N - NS * 624)])


_sc_scatter = pl.kernel(
    _sc_body,
    out_type=jax.ShapeDtypeStruct((NC, N, EMB), jnp.float32),
    mesh=plsc.VectorSubcoreMesh(
        core_axis_name="c", subcore_axis_name="s",
        num_cores=NC, num_subcores=NS),
    scratch_types=[
        pltpu.VMEM((2 * MEGA,), jnp.int32),
        pltpu.VMEM((2 * MEGA,), jnp.int32),
        pltpu.VMEM((2 * MEGA,), jnp.int32),
        pltpu.VMEM((2, NSM, SUB), jnp.int32),
        pltpu.VMEM((2, NSM, SUB), jnp.int32),
        pltpu.VMEM((2, NSM, SUB), jnp.int32),
        pltpu.VMEM((3, SUB, EMB), jnp.float32),
        pltpu.VMEM((3, SUB), jnp.float32),
        pltpu.VMEM((25, EMB), jnp.float32),
        pltpu.VMEM_SHARED((CPAD,), jnp.float32),
        pltpu.VMEM_SHARED((N, EMB), jnp.float32),
    ] + [pltpu.SemaphoreType.DMA] * 12,
)


def _comb_body(p_ref, o_ref):
    o_ref[...] = jnp.maximum(p_ref[0] + p_ref[1], 0.0)


_comb = pl.pallas_call(
    _comb_body,
    grid=(N // _BN,),
    in_specs=[pl.BlockSpec((NC, _BN, EMB), lambda i: (0, i, 0))],
    out_specs=pl.BlockSpec((_BN, EMB), lambda i: (i, 0)),
    out_shape=jax.ShapeDtypeStruct((N, EMB), jnp.float32),
)


def kernel(x, weights, edge_src, edge_dst, edge_rel):
    edge_src = edge_src.astype(jnp.int32)
    edge_dst = edge_dst.astype(jnp.int32)
    edge_rel = edge_rel.astype(jnp.int32)
    counts = _sc_counts(edge_dst, edge_rel)
    y = _mm(x, weights).reshape(R * N, EMB)
    partial = _sc_scatter(y, edge_src, edge_dst, edge_rel, counts)
    return _comb(partial)
